# baseline (speedup 1.0000x reference)
import functools

import jax
import jax.numpy as jnp
from jax import lax
from jax.experimental import pallas as pl
from jax.experimental.pallas import tpu as pltpu
from jax.experimental.pallas import tpu_sc as plsc

NUM_EXPERTS = 64
TOP_K = 8
BLK = 2048

_info = plsc.get_sparse_core_info()
NC, NS, L = _info.num_cores, _info.num_subcores, _info.num_lanes
NW = NC * NS


def _logits_block(x_ref, w_ref, lt_ref):
    x = x_ref[...]
    w = w_ref[...]
    logits = jax.lax.dot_general(
        x, w,
        dimension_numbers=(((1,), (1,)), ((), ())),
        preferred_element_type=jnp.float32,
    )
    lt_ref[...] = logits.T


def _sc_topk(logits_hbm, idx_hbm, val_hbm, lt_v, idx_v, val_v):
    n = logits_hbm.shape[1]
    ntok = n // NW
    wid = lax.axis_index("s") * NC + lax.axis_index("c")
    base = wid * ntok
    pltpu.sync_copy(logits_hbm.at[:, pl.ds(base, ntok)], lt_v)

    def group(g, _):
        def step(e, carry):
            vs, ids = carry
            x = lt_v[e, pl.ds(g * L, L)]
            xi = jnp.full((L,), 0, jnp.int32) + e
            new_vs, new_ids = [], []
            for r in range(TOP_K):
                gt = x > vs[r]
                new_vs.append(jnp.where(gt, x, vs[r]))
                new_ids.append(jnp.where(gt, xi, ids[r]))
                x = jnp.where(gt, vs[r], x)
                xi = jnp.where(gt, ids[r], xi)
            return tuple(new_vs), tuple(new_ids)

        init = (
            tuple(jnp.full((L,), -jnp.inf, jnp.float32) for _ in range(TOP_K)),
            tuple(jnp.full((L,), 0, jnp.int32) for _ in range(TOP_K)),
        )
        vs, ids = lax.fori_loop(0, NUM_EXPERTS, step, init)
        es = [jnp.exp(v - vs[0]) for v in vs]
        s = es[0]
        for r in range(1, TOP_K):
            s = s + es[r]
        for r in range(TOP_K):
            idx_v[r, pl.ds(g * L, L)] = ids[r]
            val_v[r, pl.ds(g * L, L)] = es[r] / s
        return 0

    lax.fori_loop(0, ntok // L, group, 0)
    pltpu.sync_copy(idx_v, idx_hbm.at[:, pl.ds(base, ntok)])
    pltpu.sync_copy(val_v, val_hbm.at[:, pl.ds(base, ntok)])


@functools.partial(jax.jit, static_argnames=())
def kernel(hidden_states, W):
    b, s, d = hidden_states.shape
    n = b * s
    flat = hidden_states.reshape(n, d)
    logits_t = pl.pallas_call(
        _logits_block,
        grid=(n // BLK,),
        in_specs=[
            pl.BlockSpec((BLK, d), lambda i: (i, 0)),
            pl.BlockSpec((NUM_EXPERTS, d), lambda i: (0, 0)),
        ],
        out_specs=pl.BlockSpec((NUM_EXPERTS, BLK), lambda i: (0, i)),
        out_shape=jax.ShapeDtypeStruct((NUM_EXPERTS, n), jnp.float32),
    )(flat, W)

    ntok = n // NW
    mesh = plsc.VectorSubcoreMesh(core_axis_name="c", subcore_axis_name="s")
    idx_t, val_t = pl.kernel(
        _sc_topk,
        mesh=mesh,
        out_type=[
            jax.ShapeDtypeStruct((TOP_K, n), jnp.int32),
            jax.ShapeDtypeStruct((TOP_K, n), jnp.float32),
        ],
        scratch_types=[
            pltpu.VMEM((NUM_EXPERTS, ntok), jnp.float32),
            pltpu.VMEM((TOP_K, ntok), jnp.int32),
            pltpu.VMEM((TOP_K, ntok), jnp.float32),
        ],
    )(logits_t)
    return idx_t.T, val_t.T


# speedup vs baseline: 1.3307x; 1.3307x over previous
import functools

import jax
import jax.numpy as jnp
from jax.experimental import pallas as pl
from jax.experimental.pallas import tpu as pltpu

NUM_EXPERTS = 64
TOP_K = 8
BLK = 2048


def _router_block(x_ref, w_ref, idx_ref, val_ref):
    x = x_ref[...]
    w = w_ref[...]
    logits = jax.lax.dot_general(
        x, w,
        dimension_numbers=(((1,), (1,)), ((), ())),
        preferred_element_type=jnp.float32,
    )

    blk = logits.shape[0]
    lt = logits.T
    sub_e = jax.lax.broadcasted_iota(jnp.int32, (NUM_EXPERTS, blk), 0)
    sub_k = jax.lax.broadcasted_iota(jnp.int32, (TOP_K, blk), 0)

    idx_out = jnp.zeros((TOP_K, blk), jnp.int32)
    val_out = jnp.zeros((TOP_K, blk), jnp.float32)
    for k in range(TOP_K):
        m = jnp.max(lt, axis=0, keepdims=True)
        sel = jnp.min(jnp.where(lt == m, sub_e, NUM_EXPERTS),
                      axis=0, keepdims=True)
        idx_out = jnp.where(sub_k == k, sel, idx_out)
        val_out = jnp.where(sub_k == k, m, val_out)
        lt = jnp.where(sub_e == sel, -jnp.inf, lt)

    e = jnp.exp(val_out - val_out[:1])
    w_out = e / jnp.sum(e, axis=0, keepdims=True)
    idx_ref[...] = idx_out.T
    val_ref[...] = w_out.T


@functools.partial(jax.jit, static_argnames=())
def kernel(hidden_states, W):
    b, s, d = hidden_states.shape
    n = b * s
    flat = hidden_states.reshape(n, d)
    grid = (n // BLK,)
    out = pl.pallas_call(
        _router_block,
        grid=grid,
        in_specs=[
            pl.BlockSpec((BLK, d), lambda i: (i, 0)),
            pl.BlockSpec((NUM_EXPERTS, d), lambda i: (0, 0)),
        ],
        out_specs=[
            pl.BlockSpec((BLK, TOP_K), lambda i: (i, 0)),
            pl.BlockSpec((BLK, TOP_K), lambda i: (i, 0)),
        ],
        out_shape=[
            jax.ShapeDtypeStruct((n, TOP_K), jnp.int32),
            jax.ShapeDtypeStruct((n, TOP_K), jnp.float32),
        ],
        compiler_params=pltpu.CompilerParams(
            dimension_semantics=("parallel",),
        ),
    )(flat, W)
    return out[0], out[1]
